# baseline (speedup 1.0000x reference)
import functools

import jax
import jax.numpy as jnp
from jax import lax
from jax.experimental import pallas as pl
from jax.experimental.pallas import tpu as pltpu
from jax.experimental.pallas import tpu_sc as plsc

VOCAB = 100000
EMBED = 128
CONTEXT = 200
HIDDEN = 128

B_PER_W = 16
NW_USED = 16
B_PAD = B_PER_W * NW_USED

K_BLK = 3200
NK = (CONTEXT * EMBED) // K_BLK
V_BLK = 8192
NV = -(-VOCAB // V_BLK)
P1 = NK
P2 = NK + NV
NSTEPS = NK + 2 * NV


def _gather_sc(idx_pad, table):
    mesh = plsc.VectorSubcoreMesh(core_axis_name="c", subcore_axis_name="s",
                                  num_cores=1)

    @functools.partial(
        pl.kernel,
        out_type=jax.ShapeDtypeStruct((B_PAD, EMBED), jnp.float32),
        mesh=mesh,
        scratch_types=[
            pltpu.VMEM((B_PER_W,), jnp.int32),
            pltpu.VMEM((B_PER_W, EMBED), jnp.float32),
            pltpu.SemaphoreType.DMA,
        ],
    )
    def gather(idx_hbm, table_hbm, out_hbm, idx_v, rows_v, sem):
        wid = lax.axis_index("s")
        base = wid * B_PER_W
        pltpu.sync_copy(idx_hbm.at[pl.ds(base, B_PER_W)], idx_v)
        pltpu.async_copy(table_hbm.at[idx_v], rows_v, sem).wait()
        pltpu.sync_copy(rows_v, out_hbm.at[pl.ds(base, B_PER_W)])

    return gather(idx_pad, table)


def _fused_tc(e_flat, W1, b1_row, W2, b2_row):
    def body(e_ref, w1_ref, b1_ref, w2_ref, b2_ref, out_ref,
             acc_ref, logits_ref, m_ref, s_ref):
        i = pl.program_id(0)

        @pl.when(i < P1)
        def _():
            @pl.when(i == 0)
            def _():
                acc_ref[...] = jnp.zeros_like(acc_ref)

            acc_ref[...] += jnp.dot(e_ref[...], w1_ref[...],
                                    preferred_element_type=jnp.float32)

            @pl.when(i == P1 - 1)
            def _():
                acc_ref[...] = jnp.maximum(acc_ref[...] + b1_ref[...], 0.0)

        @pl.when((i >= P1) & (i < P2))
        def _():
            j = i - P1
            z = jnp.dot(acc_ref[...], w2_ref[...],
                        preferred_element_type=jnp.float32) + b2_ref[...]
            logits_ref[pl.ds(j, 1), :] = z
            col = j * V_BLK + lax.broadcasted_iota(jnp.int32, (1, V_BLK), 1)
            zm = jnp.where(col < VOCAB, z, -jnp.inf)
            bm = jnp.max(zm)

            @pl.when(j == 0)
            def _():
                m_ref[0] = bm
                s_ref[0] = jnp.sum(jnp.exp(zm - bm))

            @pl.when(j > 0)
            def _():
                m_old = m_ref[0]
                new_m = jnp.maximum(m_old, bm)
                s_ref[0] = s_ref[0] * jnp.exp(m_old - new_m) + \
                    jnp.sum(jnp.exp(zm - new_m))
                m_ref[0] = new_m

        @pl.when(i >= P2)
        def _():
            j = i - P2
            norm = m_ref[0] + jnp.log(s_ref[0])
            out_ref[...] = logits_ref[pl.ds(j, 1), :] - norm

    return pl.pallas_call(
        body,
        grid=(NSTEPS,),
        in_specs=[
            pl.BlockSpec((1, K_BLK),
                         lambda i: (0, jnp.minimum(i, P1 - 1))),
            pl.BlockSpec((K_BLK, HIDDEN),
                         lambda i: (jnp.minimum(i, P1 - 1), 0)),
            pl.BlockSpec((1, HIDDEN), lambda i: (0, 0)),
            pl.BlockSpec((HIDDEN, V_BLK),
                         lambda i: (0, jnp.clip(i - P1, 0, NV - 1))),
            pl.BlockSpec((1, V_BLK),
                         lambda i: (0, jnp.clip(i - P1, 0, NV - 1))),
        ],
        out_specs=pl.BlockSpec((1, V_BLK),
                               lambda i: (0, jnp.clip(i - P2, 0, NV - 1))),
        out_shape=jax.ShapeDtypeStruct((1, VOCAB), jnp.float32),
        scratch_shapes=[
            pltpu.VMEM((1, HIDDEN), jnp.float32),
            pltpu.VMEM((NV, V_BLK), jnp.float32),
            pltpu.SMEM((1,), jnp.float32),
            pltpu.SMEM((1,), jnp.float32),
        ],
    )(e_flat, W1, b1_row, W2, b2_row)


def kernel(idx, table, W1, b1, W2, b2):
    idx_pad = jnp.zeros((B_PAD,), jnp.int32).at[:CONTEXT].set(
        idx.astype(jnp.int32))
    rows = _gather_sc(idx_pad, table)
    e_flat = rows[:CONTEXT].reshape(1, CONTEXT * EMBED)
    return _fused_tc(e_flat, W1, b1.reshape(1, HIDDEN),
                     W2, b2.reshape(1, VOCAB))


# speedup vs baseline: 1.1732x; 1.1732x over previous
import jax
import jax.numpy as jnp
from jax import lax
from jax.experimental import pallas as pl
from jax.experimental.pallas import tpu as pltpu

VOCAB = 100000
EMBED = 128
CONTEXT = 200
HIDDEN = 128

K_BLK = 3200
NK = (CONTEXT * EMBED) // K_BLK
V_BLK = 8192
NV = -(-VOCAB // V_BLK)
P1 = NK
P2 = NK + NV
NSTEPS = NK + 2 * NV


def _fused(idx, table, W1, b1_row, W2, b2_row):
    def body(idx_ref, table_ref, w1_ref, b1_ref, w2_ref, b2_ref, out_ref,
             e_ref, acc_ref, logits_ref, m_ref, s_ref, gsem):
        i = pl.program_id(0)

        @pl.when(i == 0)
        def _():
            def issue(j, _):
                row = idx_ref[j]
                pltpu.make_async_copy(
                    table_ref.at[pl.ds(row, 1), :],
                    e_ref.at[0:1, pl.ds(pl.multiple_of(j * EMBED, 128),
                                        EMBED)],
                    gsem).start()
                return 0
            lax.fori_loop(0, CONTEXT, issue, 0)

            def drain(j, _):
                row = idx_ref[j]
                pltpu.make_async_copy(
                    table_ref.at[pl.ds(row, 1), :],
                    e_ref.at[0:1, pl.ds(pl.multiple_of(j * EMBED, 128),
                                        EMBED)],
                    gsem).wait()
                return 0
            lax.fori_loop(0, CONTEXT, drain, 0)

        @pl.when(i < P1)
        def _():
            @pl.when(i == 0)
            def _():
                acc_ref[...] = jnp.zeros_like(acc_ref)

            e_blk = e_ref[0:1, pl.ds(pl.multiple_of(i * K_BLK, 128), K_BLK)]
            acc_ref[...] += jnp.dot(e_blk, w1_ref[...],
                                    preferred_element_type=jnp.float32)

            @pl.when(i == P1 - 1)
            def _():
                acc_ref[...] = jnp.maximum(acc_ref[...] + b1_ref[...], 0.0)

        @pl.when((i >= P1) & (i < P2))
        def _():
            j = i - P1
            z = jnp.dot(acc_ref[...], w2_ref[...],
                        preferred_element_type=jnp.float32) + b2_ref[...]
            logits_ref[pl.ds(j, 1), :] = z
            col = j * V_BLK + lax.broadcasted_iota(jnp.int32, (1, V_BLK), 1)
            zm = jnp.where(col < VOCAB, z, -jnp.inf)
            bm = jnp.max(zm)

            @pl.when(j == 0)
            def _():
                m_ref[0] = bm
                s_ref[0] = jnp.sum(jnp.exp(zm - bm))

            @pl.when(j > 0)
            def _():
                m_old = m_ref[0]
                new_m = jnp.maximum(m_old, bm)
                s_ref[0] = s_ref[0] * jnp.exp(m_old - new_m) + \
                    jnp.sum(jnp.exp(zm - new_m))
                m_ref[0] = new_m

        @pl.when(i >= P2)
        def _():
            j = i - P2
            norm = m_ref[0] + jnp.log(s_ref[0])
            out_ref[...] = logits_ref[pl.ds(j, 1), :] - norm

    return pl.pallas_call(
        body,
        grid=(NSTEPS,),
        in_specs=[
            pl.BlockSpec(memory_space=pltpu.SMEM),
            pl.BlockSpec(memory_space=pl.ANY),
            pl.BlockSpec((K_BLK, HIDDEN),
                         lambda i: (jnp.minimum(i, P1 - 1), 0)),
            pl.BlockSpec((1, HIDDEN), lambda i: (0, 0)),
            pl.BlockSpec((HIDDEN, V_BLK),
                         lambda i: (0, jnp.clip(i - P1, 0, NV - 1))),
            pl.BlockSpec((1, V_BLK),
                         lambda i: (0, jnp.clip(i - P1, 0, NV - 1))),
        ],
        out_specs=pl.BlockSpec((1, V_BLK),
                               lambda i: (0, jnp.clip(i - P2, 0, NV - 1))),
        out_shape=jax.ShapeDtypeStruct((1, VOCAB), jnp.float32),
        scratch_shapes=[
            pltpu.VMEM((1, CONTEXT * EMBED), jnp.float32),
            pltpu.VMEM((1, HIDDEN), jnp.float32),
            pltpu.VMEM((NV, V_BLK), jnp.float32),
            pltpu.SMEM((1,), jnp.float32),
            pltpu.SMEM((1,), jnp.float32),
            pltpu.SemaphoreType.DMA,
        ],
    )(idx, table, W1, b1_row, W2, b2_row)


def kernel(idx, table, W1, b1, W2, b2):
    return _fused(idx.astype(jnp.int32), table, W1, b1.reshape(1, -1),
                  W2, b2.reshape(1, -1))


# speedup vs baseline: 2.6062x; 2.2214x over previous
import jax
import jax.numpy as jnp
from jax import lax
from jax.experimental import pallas as pl
from jax.experimental.pallas import tpu as pltpu

VOCAB = 100000
EMBED = 128
CONTEXT = 200
HIDDEN = 128

K_BLK = 3200
NK = (CONTEXT * EMBED) // K_BLK
V_BLK = 8192
NV = -(-VOCAB // V_BLK)
P1 = NK
P2 = NK + NV
NSTEPS = NK + 2 * NV


def _fused(idx, table, W1, b1, W2T, b2):
    def body(idx_ref, table_ref, w1_ref, b1_ref, w2_ref, b2_ref, out_ref,
             e_ref, acc_ref, logits_ref, m_ref, s_ref, gsem):
        i = pl.program_id(0)

        @pl.when(i == 0)
        def _():
            def issue(j, _):
                row = idx_ref[j]
                pltpu.make_async_copy(
                    table_ref.at[pl.ds(row, 1), :],
                    e_ref.at[0:1, pl.ds(pl.multiple_of(j * EMBED, 128),
                                        EMBED)],
                    gsem).start()
                return 0
            lax.fori_loop(0, CONTEXT, issue, 0)

            def drain(j, _):
                row = idx_ref[j]
                pltpu.make_async_copy(
                    table_ref.at[pl.ds(row, 1), :],
                    e_ref.at[0:1, pl.ds(pl.multiple_of(j * EMBED, 128),
                                        EMBED)],
                    gsem).wait()
                return 0
            lax.fori_loop(0, CONTEXT, drain, 0)

        @pl.when(i < P1)
        def _():
            @pl.when(i == 0)
            def _():
                acc_ref[...] = jnp.zeros_like(acc_ref)

            e_blk = e_ref[0:1, pl.ds(pl.multiple_of(i * K_BLK, 128), K_BLK)]
            acc_ref[...] += jnp.dot(e_blk, w1_ref[...],
                                    preferred_element_type=jnp.float32)

            @pl.when(i == P1 - 1)
            def _():
                acc_ref[...] = jnp.maximum(acc_ref[...] + b1_ref[...], 0.0)

        @pl.when((i >= P1) & (i < P2))
        def _():
            j = i - P1
            z = lax.dot_general(acc_ref[...], w2_ref[...],
                                (((1,), (1,)), ((), ())),
                                preferred_element_type=jnp.float32) \
                + b2_ref[...]
            logits_ref[pl.ds(j, 1), :] = z
            col = j * V_BLK + lax.broadcasted_iota(jnp.int32, (1, V_BLK), 1)
            zm = jnp.where(col < VOCAB, z, -jnp.inf)
            bm = jnp.max(zm)

            @pl.when(j == 0)
            def _():
                m_ref[0] = bm
                s_ref[0] = jnp.sum(jnp.exp(zm - bm))

            @pl.when(j > 0)
            def _():
                m_old = m_ref[0]
                new_m = jnp.maximum(m_old, bm)
                s_ref[0] = s_ref[0] * jnp.exp(m_old - new_m) + \
                    jnp.sum(jnp.exp(zm - new_m))
                m_ref[0] = new_m

        @pl.when(i >= P2)
        def _():
            j = i - P2
            norm = m_ref[0] + jnp.log(s_ref[0])
            out_ref[...] = logits_ref[pl.ds(j, 1), :] - norm

    return pl.pallas_call(
        body,
        grid=(NSTEPS,),
        in_specs=[
            pl.BlockSpec(memory_space=pltpu.SMEM),
            pl.BlockSpec(memory_space=pl.ANY),
            pl.BlockSpec((K_BLK, HIDDEN),
                         lambda i: (jnp.minimum(i, P1 - 1), 0)),
            pl.BlockSpec((HIDDEN,), lambda i: (0,)),
            pl.BlockSpec((V_BLK, HIDDEN),
                         lambda i: (jnp.clip(i - P1, 0, NV - 1), 0)),
            pl.BlockSpec((V_BLK,),
                         lambda i: (jnp.clip(i - P1, 0, NV - 1),)),
        ],
        out_specs=pl.BlockSpec((1, V_BLK),
                               lambda i: (0, jnp.clip(i - P2, 0, NV - 1))),
        out_shape=jax.ShapeDtypeStruct((1, VOCAB), jnp.float32),
        scratch_shapes=[
            pltpu.VMEM((1, CONTEXT * EMBED), jnp.float32),
            pltpu.VMEM((1, HIDDEN), jnp.float32),
            pltpu.VMEM((NV, V_BLK), jnp.float32),
            pltpu.SMEM((1,), jnp.float32),
            pltpu.SMEM((1,), jnp.float32),
            pltpu.SemaphoreType.DMA,
        ],
    )(idx, table, W1, b1, W2T, b2)


def kernel(idx, table, W1, b1, W2, b2):
    return _fused(idx.astype(jnp.int32), table, W1, b1, W2.T, b2)


# speedup vs baseline: 2.6390x; 1.0126x over previous
import jax
import jax.numpy as jnp
from jax import lax
from jax.experimental import pallas as pl
from jax.experimental.pallas import tpu as pltpu

VOCAB = 100000
EMBED = 128
CONTEXT = 200
HIDDEN = 128

K_BLK = 3200
NK = (CONTEXT * EMBED) // K_BLK
V_BLK = 8192
NV = -(-VOCAB // V_BLK)
P1 = NK
P2 = NK + NV
NSTEPS = NK + 2 * NV


def _fused(idx, table, W1, b1, W2T, b2):
    def body(idx_ref, table_ref, w1_ref, b1_ref, w2_ref, b2_ref, out_ref,
             e_ref, acc_ref, logits_ref, m_ref, s_ref, gsem):
        i = pl.program_id(0)

        rows_per_step = CONTEXT // NK

        @pl.when(i == 0)
        def _():
            def issue(j, _):
                row = idx_ref[j]
                pltpu.make_async_copy(
                    table_ref.at[pl.ds(row, 1), :],
                    e_ref.at[0:1, pl.ds(pl.multiple_of(j * EMBED, 128),
                                        EMBED)],
                    gsem).start()
                return 0
            lax.fori_loop(0, CONTEXT, issue, 0)

        @pl.when(i < P1)
        def _():
            @pl.when(i == 0)
            def _():
                acc_ref[...] = jnp.zeros_like(acc_ref)

            def drain(j, _):
                row = idx_ref[j]
                pltpu.make_async_copy(
                    table_ref.at[pl.ds(row, 1), :],
                    e_ref.at[0:1, pl.ds(pl.multiple_of(j * EMBED, 128),
                                        EMBED)],
                    gsem).wait()
                return 0
            lax.fori_loop(i * rows_per_step, (i + 1) * rows_per_step,
                          drain, 0)

            e_blk = e_ref[0:1, pl.ds(pl.multiple_of(i * K_BLK, 128), K_BLK)]
            acc_ref[...] += jnp.dot(e_blk, w1_ref[...],
                                    preferred_element_type=jnp.float32)

            @pl.when(i == P1 - 1)
            def _():
                acc_ref[...] = jnp.maximum(acc_ref[...] + b1_ref[...], 0.0)

        @pl.when((i >= P1) & (i < P2))
        def _():
            j = i - P1
            z = lax.dot_general(acc_ref[...], w2_ref[...],
                                (((1,), (1,)), ((), ())),
                                preferred_element_type=jnp.float32) \
                + b2_ref[...]
            logits_ref[pl.ds(j, 1), :] = z
            col = j * V_BLK + lax.broadcasted_iota(jnp.int32, (1, V_BLK), 1)
            zm = jnp.where(col < VOCAB, z, -jnp.inf)
            bm = jnp.max(zm)

            @pl.when(j == 0)
            def _():
                m_ref[0] = bm
                s_ref[0] = jnp.sum(jnp.exp(zm - bm))

            @pl.when(j > 0)
            def _():
                m_old = m_ref[0]
                new_m = jnp.maximum(m_old, bm)
                s_ref[0] = s_ref[0] * jnp.exp(m_old - new_m) + \
                    jnp.sum(jnp.exp(zm - new_m))
                m_ref[0] = new_m

        @pl.when(i >= P2)
        def _():
            j = i - P2
            norm = m_ref[0] + jnp.log(s_ref[0])
            out_ref[...] = logits_ref[pl.ds(j, 1), :] - norm

    return pl.pallas_call(
        body,
        grid=(NSTEPS,),
        in_specs=[
            pl.BlockSpec(memory_space=pltpu.SMEM),
            pl.BlockSpec(memory_space=pl.ANY),
            pl.BlockSpec((K_BLK, HIDDEN),
                         lambda i: (jnp.minimum(i, P1 - 1), 0)),
            pl.BlockSpec((HIDDEN,), lambda i: (0,)),
            pl.BlockSpec((V_BLK, HIDDEN),
                         lambda i: (jnp.clip(i - P1, 0, NV - 1), 0)),
            pl.BlockSpec((V_BLK,),
                         lambda i: (jnp.clip(i - P1, 0, NV - 1),)),
        ],
        out_specs=pl.BlockSpec((1, V_BLK),
                               lambda i: (0, jnp.clip(i - P2, 0, NV - 1))),
        out_shape=jax.ShapeDtypeStruct((1, VOCAB), jnp.float32),
        scratch_shapes=[
            pltpu.VMEM((1, CONTEXT * EMBED), jnp.float32),
            pltpu.VMEM((1, HIDDEN), jnp.float32),
            pltpu.VMEM((NV, V_BLK), jnp.float32),
            pltpu.SMEM((1,), jnp.float32),
            pltpu.SMEM((1,), jnp.float32),
            pltpu.SemaphoreType.DMA,
        ],
    )(idx, table, W1, b1, W2T, b2)


def kernel(idx, table, W1, b1, W2, b2):
    return _fused(idx.astype(jnp.int32), table, W1, b1, W2.T, b2)
